# baseline (speedup 1.0000x reference)
import functools

import jax
import jax.numpy as jnp
import numpy as np
from jax import lax
from jax.experimental import pallas as pl
from jax.experimental.pallas import tpu as pltpu
from jax.experimental.pallas import tpu_sc as plsc

N = 10000
E = 320000
F = 128
NC = 2
NS = 16
LANES = 16
NW = NC * NS

CHUNK = 96
RPW = 112
BLK = 16
NBLK = RPW // BLK
NIT = BLK // 2
EROWS = NW * RPW
EPAD = EROWS * CHUNK

NPAD = 10240
ROWS_PT = NPAD // NS
ZB = 80

_fg = np.arange(F // 2) // 16
_i = np.arange(F // 2) % 16
_PERM = np.concatenate([32 * _fg + _i, 32 * _fg + 16 + _i])


def _matmul_body(x_ref, wt_ref, o_ref):
    y = jnp.dot(x_ref[...], wt_ref[...],
                preferred_element_type=jnp.float32)
    yr = y.astype(jnp.bfloat16).astype(jnp.float32)
    yi = jax.lax.bitcast_convert_type(yr, jnp.int32)
    lo = yi[:, :F // 2]
    hi = yi[:, F // 2:]
    o_ref[...] = jnp.bitwise_or(
        jnp.bitwise_and(jnp.right_shift(lo, 16), 65535),
        jnp.bitwise_and(hi, -65536))


def _combine_body(p_ref, b_ref, o_ref):
    o_ref[...] = p_ref[0] + p_ref[1] + b_ref[...][None, :]


def _scatter_body(xlinb, src_h, dst_h, w_h, out_h,
                  acc, src_v, dst_v, w_v, rowsb0, rowsb1, msg0, msg1,
                  sg0, sg1, ss0, ss1):
    c = lax.axis_index("c")
    s = lax.axis_index("s")
    w = c * NS + s

    def _zero_rows(r, _):
        for j in range(F // LANES):
            msg0[r, pl.ds(j * LANES, LANES)] = jnp.zeros(
                (LANES,), jnp.float32)
        return 0
    lax.fori_loop(0, ZB, _zero_rows, 0)
    for t in range(ROWS_PT // ZB):
        pltpu.sync_copy(msg0.at[pl.ds(0, ZB)],
                        acc.at[pl.ds(s * ROWS_PT + t * ZB, ZB)])
    plsc.subcore_barrier()

    def _scale(rowsb, msg, k):
        @plsc.parallel_loop(0, CHUNK // LANES)
        def _grp(g2):
            ebase = g2 * LANES
            wvec = w_v[k, pl.ds(ebase, LANES)]
            for l in range(LANES):
                ew = wvec[l]
                e = ebase + l
                for fg in range(F // 32):
                    v = rowsb[e, pl.ds(16 * fg, LANES)]
                    a = jax.lax.bitcast_convert_type(
                        jnp.left_shift(v, 16), jnp.float32)
                    b2 = jax.lax.bitcast_convert_type(
                        jnp.bitwise_and(v, jnp.int32(-65536)), jnp.float32)
                    msg[e, pl.ds(32 * fg, LANES)] = a * ew
                    msg[e, pl.ds(32 * fg + LANES, LANES)] = b2 * ew

    def _gather(k, rowsb, sem):
        pltpu.async_copy(xlinb.at[src_v.at[k]], rowsb, sem)

    def _gwait(k, rowsb, sem):
        pltpu.make_async_copy(xlinb.at[src_v.at[k]], rowsb, sem).wait()

    def _block(blk, _):
        brow = w * RPW + blk * BLK
        pltpu.sync_copy(src_h.at[pl.ds(brow, BLK)], src_v)
        pltpu.sync_copy(dst_h.at[pl.ds(brow, BLK)], dst_v)
        pltpu.sync_copy(w_h.at[pl.ds(brow, BLK)], w_v)

        _gather(0, rowsb0, sg0)
        _gather(1, rowsb1, sg1)

        def _iter(k2, _):
            k = 2 * k2
            _gwait(k, rowsb0, sg0)

            @pl.when(k2 > 0)
            def _():
                pltpu.make_async_copy(
                    msg0, acc.at[dst_v.at[k]], ss0).wait()
            _scale(rowsb0, msg0, k)
            pltpu.async_copy(msg0, acc.at[dst_v.at[k]], ss0, add=True)

            @pl.when(k2 < NIT - 1)
            def _():
                _gather(k + 2, rowsb0, sg0)

            _gwait(k + 1, rowsb1, sg1)

            @pl.when(k2 > 0)
            def _():
                pltpu.make_async_copy(
                    msg1, acc.at[dst_v.at[k + 1]], ss1).wait()
            _scale(rowsb1, msg1, k + 1)
            pltpu.async_copy(msg1, acc.at[dst_v.at[k + 1]], ss1, add=True)

            @pl.when(k2 < NIT - 1)
            def _():
                _gather(k + 3, rowsb1, sg1)
            return 0
        lax.fori_loop(0, NIT, _iter, 0)
        pltpu.make_async_copy(msg0, acc.at[dst_v.at[BLK - 2]], ss0).wait()
        pltpu.make_async_copy(msg1, acc.at[dst_v.at[BLK - 1]], ss1).wait()
        return 0
    lax.fori_loop(0, NBLK, _block, 0)
    plsc.subcore_barrier()

    pltpu.sync_copy(acc.at[pl.ds(s * ROWS_PT, ROWS_PT)],
                    out_h.at[c, pl.ds(s * ROWS_PT, ROWS_PT)])


_scatter_kernel = functools.partial(
    pl.kernel,
    out_type=jax.ShapeDtypeStruct((NC, NPAD, F), jnp.float32),
    mesh=plsc.VectorSubcoreMesh(core_axis_name="c", subcore_axis_name="s"),
    compiler_params=pltpu.CompilerParams(use_tc_tiling_on_sc=False),
    scratch_types=[
        pltpu.VMEM_SHARED((NPAD, F), jnp.float32),
        pltpu.VMEM((BLK, CHUNK), jnp.int32),
        pltpu.VMEM((BLK, CHUNK), jnp.int32),
        pltpu.VMEM((BLK, CHUNK), jnp.float32),
        pltpu.VMEM((CHUNK, F // 2), jnp.int32),
        pltpu.VMEM((CHUNK, F // 2), jnp.int32),
        pltpu.VMEM((CHUNK, F), jnp.float32),
        pltpu.VMEM((CHUNK, F), jnp.float32),
        pltpu.SemaphoreType.DMA,
        pltpu.SemaphoreType.DMA,
        pltpu.SemaphoreType.DMA,
        pltpu.SemaphoreType.DMA,
    ],
)(_scatter_body)


@jax.jit
def kernel(x, edge_index, edge_weight, node_lock, W, b):
    xlinb = pl.pallas_call(
        _matmul_body,
        grid=(10,),
        in_specs=[
            pl.BlockSpec((N // 10, F), lambda i: (i, 0)),
            pl.BlockSpec((F, F), lambda i: (0, 0)),
        ],
        out_specs=pl.BlockSpec((N // 10, F // 2), lambda i: (i, 0)),
        out_shape=jax.ShapeDtypeStruct((N, F // 2), jnp.int32),
    )(x, W[_PERM].T)

    npad_e = EPAD - E
    pad_idx = jnp.arange(npad_e, dtype=jnp.int32) % N
    srcp = jnp.concatenate([edge_index[0], pad_idx]).reshape(EROWS, CHUNK)
    dstp = jnp.concatenate([edge_index[1], pad_idx]).reshape(EROWS, CHUNK)
    wp = jnp.concatenate(
        [edge_weight, jnp.zeros((npad_e,), jnp.float32)]).reshape(EROWS, CHUNK)

    partials = _scatter_kernel(xlinb, srcp, dstp, wp)

    out = pl.pallas_call(
        _combine_body,
        grid=(10,),
        in_specs=[
            pl.BlockSpec((NC, N // 10, F), lambda i: (0, i, 0)),
            pl.BlockSpec((F,), lambda i: (0,)),
        ],
        out_specs=pl.BlockSpec((N // 10, F), lambda i: (i, 0)),
        out_shape=jax.ShapeDtypeStruct((N, F), jnp.float32),
    )(partials, b)

    return (out, (edge_index, edge_weight))


# speedup vs baseline: 1.5564x; 1.5564x over previous
import functools

import jax
import jax.numpy as jnp
from jax import lax
from jax.experimental import pallas as pl
from jax.experimental.pallas import tpu as pltpu
from jax.experimental.pallas import tpu_sc as plsc

N = 10000
E = 320000
F = 128
NC = 2
NS = 16
LANES = 16
NW = NC * NS

CHUNK = 112
RPW = 90
T = RPW // 3
EPAD = NW * RPW * CHUNK

NPAD = 10240
ROWS_PT = NPAD // NS
ZB = 80


def _matmul_body(x_ref, wt_ref, o_ref):
    o_ref[...] = jnp.dot(x_ref[...], wt_ref[...],
                         preferred_element_type=jnp.float32)


def _combine_body(p_ref, b_ref, o_ref):
    o_ref[...] = p_ref[0] + p_ref[1] + b_ref[...][None, :]


def _scatter_body(xlin, src_h, dst_h, w_h, out_h,
                  acc,
                  src0, src1, src2, dst0, dst1, dst2, w0, w1, w2,
                  rows0, rows1, rows2,
                  si0, si1, si2, sg0, sg1, sg2, ss0, ss1, ss2):
    c = lax.axis_index("c")
    s = lax.axis_index("s")
    w = c * NS + s
    ebase = w * RPW * CHUNK

    srcs = (src0, src1, src2)
    dsts = (dst0, dst1, dst2)
    ws = (w0, w1, w2)
    rows = (rows0, rows1, rows2)
    sis = (si0, si1, si2)
    sgs = (sg0, sg1, sg2)
    sss = (ss0, ss1, ss2)

    def _zero_rows(r, _):
        for j in range(F // LANES):
            rows0[r, pl.ds(j * LANES, LANES)] = jnp.zeros(
                (LANES,), jnp.float32)
        return 0
    lax.fori_loop(0, ZB, _zero_rows, 0)
    for t in range(ROWS_PT // ZB):
        pltpu.sync_copy(rows0.at[pl.ds(0, ZB)],
                        acc.at[pl.ds(s * ROWS_PT + t * ZB, ZB)])
    plsc.subcore_barrier()

    def _idx_start(i, k):
        eo = ebase + k * CHUNK
        pltpu.async_copy(src_h.at[pl.ds(eo, CHUNK)], srcs[i], sis[i])
        pltpu.async_copy(dst_h.at[pl.ds(eo, CHUNK)], dsts[i], sis[i])
        pltpu.async_copy(w_h.at[pl.ds(eo, CHUNK)], ws[i], sis[i])

    def _idx_wait(i, k):
        eo = ebase + k * CHUNK
        pltpu.make_async_copy(src_h.at[pl.ds(eo, CHUNK)], srcs[i],
                              sis[i]).wait()
        pltpu.make_async_copy(dst_h.at[pl.ds(eo, CHUNK)], dsts[i],
                              sis[i]).wait()
        pltpu.make_async_copy(w_h.at[pl.ds(eo, CHUNK)], ws[i],
                              sis[i]).wait()

    def _scale(i):
        rref = rows[i]
        wref = ws[i]

        @plsc.parallel_loop(0, CHUNK // LANES)
        def _grp(g2):
            eb2 = g2 * LANES
            wvec = wref[pl.ds(eb2, LANES)]
            for l in range(LANES):
                ew = wvec[l]
                e = eb2 + l
                for j in range(F // LANES):
                    sl = pl.ds(j * LANES, LANES)
                    rref[e, sl] = rref[e, sl] * ew

    for i in range(3):
        _idx_start(i, i)
    for i in range(3):
        _idx_wait(i, i)
        pltpu.async_copy(xlin.at[srcs[i]], rows[i], sgs[i])

    def _iter(t, _):
        for i in range(3):
            k = 3 * t + i

            @pl.when(t > 0)
            def _():
                pltpu.make_async_copy(rows[i], acc.at[dsts[i]],
                                      sss[i]).wait()

            @pl.when(t < T - 1)
            def _():
                _idx_start(i, k + 3)

            pltpu.make_async_copy(xlin.at[srcs[i]], rows[i], sgs[i]).wait()
            _scale(i)
            pltpu.async_copy(rows[i], acc.at[dsts[i]], sss[i], add=True)

            @pl.when(t < T - 1)
            def _():
                _idx_wait(i, k + 3)
                pltpu.async_copy(xlin.at[srcs[i]], rows[i], sgs[i])
        return 0
    lax.fori_loop(0, T, _iter, 0)
    for i in range(3):
        pltpu.make_async_copy(rows[i], acc.at[dsts[i]], sss[i]).wait()
    plsc.subcore_barrier()

    pltpu.sync_copy(acc.at[pl.ds(s * ROWS_PT, ROWS_PT)],
                    out_h.at[c, pl.ds(s * ROWS_PT, ROWS_PT)])


_scatter_kernel = functools.partial(
    pl.kernel,
    out_type=jax.ShapeDtypeStruct((NC, NPAD, F), jnp.float32),
    mesh=plsc.VectorSubcoreMesh(core_axis_name="c", subcore_axis_name="s"),
    scratch_types=(
        [pltpu.VMEM_SHARED((NPAD, F), jnp.float32)]
        + [pltpu.VMEM((CHUNK,), jnp.int32) for _ in range(6)]
        + [pltpu.VMEM((CHUNK,), jnp.float32) for _ in range(3)]
        + [pltpu.VMEM((CHUNK, F), jnp.float32) for _ in range(3)]
        + [pltpu.SemaphoreType.DMA for _ in range(9)]
    ),
)(_scatter_body)


@jax.jit
def kernel(x, edge_index, edge_weight, node_lock, W, b):
    x_lin = pl.pallas_call(
        _matmul_body,
        grid=(10,),
        in_specs=[
            pl.BlockSpec((N // 10, F), lambda i: (i, 0)),
            pl.BlockSpec((F, F), lambda i: (0, 0)),
        ],
        out_specs=pl.BlockSpec((N // 10, F), lambda i: (i, 0)),
        out_shape=jax.ShapeDtypeStruct((N, F), jnp.float32),
    )(x, W.T)

    npad_e = EPAD - E
    pad_idx = jnp.arange(npad_e, dtype=jnp.int32) % N
    srcp = jnp.concatenate([edge_index[0], pad_idx])
    dstp = jnp.concatenate([edge_index[1], pad_idx])
    wp = jnp.concatenate([edge_weight, jnp.zeros((npad_e,), jnp.float32)])

    partials = _scatter_kernel(x_lin, srcp, dstp, wp)

    out = pl.pallas_call(
        _combine_body,
        grid=(10,),
        in_specs=[
            pl.BlockSpec((NC, N // 10, F), lambda i: (0, i, 0)),
            pl.BlockSpec((F,), lambda i: (0,)),
        ],
        out_specs=pl.BlockSpec((N // 10, F), lambda i: (i, 0)),
        out_shape=jax.ShapeDtypeStruct((N, F), jnp.float32),
    )(partials, b)

    return (out, (edge_index, edge_weight))
